# Initial kernel scaffold; baseline (speedup 1.0000x reference)
import jax
import jax.numpy as jnp
from jax.experimental import pallas as pl


def kernel(x_input, semantic_feature, adj1, adj2, gc_w1, gc_w2, fc_w, fc_b, cls_w, cls_b):
    raise NotImplementedError("write your pallas kernel here")


# speedup vs baseline: 1.6968x; 1.6968x over previous
import jax
import jax.numpy as jnp
from jax.experimental import pallas as pl
from jax.experimental.pallas import tpu as pltpu

_G = 4
_OPDT = jnp.bfloat16


def _gcn_kernel(x_ref, ah1_ref, ah2_ref, w1_ref, w2_ref,
                fcw_ref, fcb_ref, clsw_ref, clsb_ref, out_ref):
    GC = x_ref.shape[0]
    C = ah1_ref.shape[1]
    G = GC // C
    Do = w2_ref.shape[1]
    f32 = jnp.float32
    cdt = w1_ref.dtype

    x = x_ref[...]

    s1 = jnp.dot(x, w1_ref[...], preferred_element_type=f32)
    s1 = s1.astype(cdt)

    a1 = ah1_ref[...]
    h1 = jnp.concatenate(
        [jnp.dot(a1, s1[g * C:(g + 1) * C], preferred_element_type=f32)
         for g in range(G)], axis=0)
    h1 = jnp.maximum(h1, 0.2 * h1)

    s2 = jnp.dot(h1.astype(cdt), w2_ref[...],
                 preferred_element_type=f32).astype(cdt)

    a2 = ah2_ref[...]
    f1s, f2s = [], []
    for g in range(G):
        gg = jnp.dot(a2, s2[g * 2 * C:(g + 1) * 2 * C],
                     preferred_element_type=f32)
        f1s.append(gg[:C])
        f2s.append(gg[C:])
    f1 = jnp.concatenate(f1s, axis=0).astype(cdt)
    f2 = jnp.concatenate(f2s, axis=0).astype(cdt)

    pre = (jnp.dot(f1, fcw_ref[0:Do, :], preferred_element_type=f32)
           + jnp.dot(f2, fcw_ref[Do:2 * Do, :], preferred_element_type=f32)
           + jnp.dot(x, fcw_ref[2 * Do:, :], preferred_element_type=f32)
           + fcb_ref[...])
    out = jnp.tanh(pre)

    out_ref[...] = (jnp.sum(out * clsw_ref[...], axis=-1, keepdims=True)
                    + clsb_ref[...])


def kernel(x_input, semantic_feature, adj1, adj2, gc_w1, gc_w2,
           fc_w, fc_b, cls_w, cls_b):
    B, C, D = semantic_feature.shape
    Dm = gc_w1.shape[1]
    Do = gc_w2.shape[1]
    Dout = fc_w.shape[0]
    G = _G
    f32 = jnp.float32
    cdt = _OPDT

    sem_flat = semantic_feature.reshape(B * C, D).astype(cdt)

    a1f = adj1.astype(f32)
    a2f = adj2.astype(f32)
    a_hop1 = jnp.concatenate([a1f, a2f], axis=0).astype(cdt)
    zeros = jnp.zeros((C, C), f32)
    a_hop2 = jnp.concatenate(
        [jnp.concatenate([a1f, zeros], axis=1),
         jnp.concatenate([zeros, a2f], axis=1)], axis=0).astype(cdt)

    fcw = fc_w.T.astype(cdt)
    fcb = fc_b.reshape(1, Dout).astype(f32)
    clsw = jnp.tile(cls_w.astype(f32), (G, 1))
    clsb = jnp.tile(cls_b.reshape(C, 1).astype(f32), (G, 1))
    w1 = gc_w1.astype(cdt)
    w2 = gc_w2.astype(cdt)

    out = pl.pallas_call(
        _gcn_kernel,
        out_shape=jax.ShapeDtypeStruct((B * C, 1), f32),
        grid_spec=pltpu.PrefetchScalarGridSpec(
            num_scalar_prefetch=0,
            grid=(B // G,),
            in_specs=[
                pl.BlockSpec((G * C, D), lambda b: (b, 0)),
                pl.BlockSpec((2 * C, C), lambda b: (0, 0)),
                pl.BlockSpec((2 * C, 2 * C), lambda b: (0, 0)),
                pl.BlockSpec((D, Dm), lambda b: (0, 0)),
                pl.BlockSpec((Dm, Do), lambda b: (0, 0)),
                pl.BlockSpec((2 * Do + D, Dout), lambda b: (0, 0)),
                pl.BlockSpec((1, Dout), lambda b: (0, 0)),
                pl.BlockSpec((G * C, Dout), lambda b: (0, 0)),
                pl.BlockSpec((G * C, 1), lambda b: (0, 0)),
            ],
            out_specs=pl.BlockSpec((G * C, 1), lambda b: (b, 0)),
        ),
        compiler_params=pltpu.CompilerParams(
            dimension_semantics=("parallel",),
            vmem_limit_bytes=56 << 20,
        ),
    )(sem_flat, a_hop1, a_hop2, w1, w2, fcw, fcb, clsw, clsb)

    return out.reshape(B, C)


# speedup vs baseline: 2.1495x; 1.2668x over previous
import jax
import jax.numpy as jnp
from jax.experimental import pallas as pl
from jax.experimental.pallas import tpu as pltpu

_G = 4
_SPLIT = 2


def _gcn_kernel(x_ref, ah1_ref, ah2_ref, w1_ref, w2_ref,
                fcw_ref, fcb_ref, clsw_ref, clsb_ref, out_ref,
                w1s, w2s, fcws):
    GC = x_ref.shape[0]
    C = ah1_ref.shape[1]
    G = GC // C
    Do = w2_ref.shape[1]
    f32 = jnp.float32
    cdt = jnp.bfloat16

    @pl.when(pl.program_id(1) == 0)
    def _prep():
        w1s[...] = w1_ref[...].astype(cdt)
        w2s[...] = w2_ref[...].astype(cdt)
        fcws[...] = fcw_ref[...].astype(cdt)

    x = x_ref[...].astype(cdt)

    s1 = jnp.dot(x, w1s[...], preferred_element_type=f32)
    s1 = s1.astype(cdt)

    a1 = ah1_ref[...]
    h1 = jnp.concatenate(
        [jnp.dot(a1, s1[g * C:(g + 1) * C], preferred_element_type=f32)
         for g in range(G)], axis=0)
    h1 = jnp.maximum(h1, 0.2 * h1)

    s2 = jnp.dot(h1.astype(cdt), w2s[...],
                 preferred_element_type=f32).astype(cdt)

    a2 = ah2_ref[...]
    f1s, f2s = [], []
    for g in range(G):
        gg = jnp.dot(a2, s2[g * 2 * C:(g + 1) * 2 * C],
                     preferred_element_type=f32)
        f1s.append(gg[:C])
        f2s.append(gg[C:])
    f1 = jnp.concatenate(f1s, axis=0).astype(cdt)
    f2 = jnp.concatenate(f2s, axis=0).astype(cdt)

    dn = (((1,), (1,)), ((), ()))
    fcw = fcws[...]
    pre = (jax.lax.dot_general(f1, fcw[:, 0:Do], dn, preferred_element_type=f32)
           + jax.lax.dot_general(f2, fcw[:, Do:2 * Do], dn, preferred_element_type=f32)
           + jax.lax.dot_general(x, fcw[:, 2 * Do:], dn, preferred_element_type=f32)
           + fcb_ref[...])
    out = jnp.tanh(pre)

    out_ref[...] = (jnp.sum(out * clsw_ref[...], axis=-1, keepdims=True)
                    + clsb_ref[...])


def kernel(x_input, semantic_feature, adj1, adj2, gc_w1, gc_w2,
           fc_w, fc_b, cls_w, cls_b):
    B, C, D = semantic_feature.shape
    Dm = gc_w1.shape[1]
    Do = gc_w2.shape[1]
    Dout = fc_w.shape[0]
    G = _G
    S = _SPLIT
    J = B // G // S
    f32 = jnp.float32
    cdt = jnp.bfloat16

    sem_flat = semantic_feature.reshape(B * C, D)

    a1f = adj1.astype(f32)
    a2f = adj2.astype(f32)
    a_hop1 = jnp.concatenate([a1f, a2f], axis=0).astype(cdt)
    zeros = jnp.zeros((C, C), f32)
    a_hop2 = jnp.concatenate(
        [jnp.concatenate([a1f, zeros], axis=1),
         jnp.concatenate([zeros, a2f], axis=1)], axis=0).astype(cdt)

    fcb = fc_b.reshape(1, Dout)
    clsw = jnp.tile(cls_w, (G, 1))
    clsb = jnp.tile(cls_b.reshape(C, 1), (G, 1))

    out = pl.pallas_call(
        _gcn_kernel,
        out_shape=jax.ShapeDtypeStruct((B * C, 1), f32),
        grid_spec=pltpu.PrefetchScalarGridSpec(
            num_scalar_prefetch=0,
            grid=(S, J),
            in_specs=[
                pl.BlockSpec((G * C, D), lambda i, j: (i * J + j, 0)),
                pl.BlockSpec((2 * C, C), lambda i, j: (0, 0)),
                pl.BlockSpec((2 * C, 2 * C), lambda i, j: (0, 0)),
                pl.BlockSpec((D, Dm), lambda i, j: (0, 0)),
                pl.BlockSpec((Dm, Do), lambda i, j: (0, 0)),
                pl.BlockSpec((Dout, 2 * Do + D), lambda i, j: (0, 0)),
                pl.BlockSpec((1, Dout), lambda i, j: (0, 0)),
                pl.BlockSpec((G * C, Dout), lambda i, j: (0, 0)),
                pl.BlockSpec((G * C, 1), lambda i, j: (0, 0)),
            ],
            out_specs=pl.BlockSpec((G * C, 1), lambda i, j: (i * J + j, 0)),
            scratch_shapes=[
                pltpu.VMEM((D, Dm), cdt),
                pltpu.VMEM((Dm, Do), cdt),
                pltpu.VMEM((Dout, 2 * Do + D), cdt),
            ],
        ),
        compiler_params=pltpu.CompilerParams(
            dimension_semantics=("parallel", "arbitrary"),
            vmem_limit_bytes=60 << 20,
        ),
    )(sem_flat, a_hop1, a_hop2, gc_w1, gc_w2, fc_w, fcb, clsw, clsb)

    return out.reshape(B, C)


# speedup vs baseline: 2.1722x; 1.0106x over previous
import jax
import jax.numpy as jnp
from jax.experimental import pallas as pl
from jax.experimental.pallas import tpu as pltpu

_G = 4
_SPLIT = 1


def _gcn_kernel(x_ref, ah1_ref, ah2_ref, w1_ref, w2_ref,
                fcw_ref, fcb_ref, clsw_ref, clsb_ref, out_ref,
                w1s, w2s, fcws):
    GC = x_ref.shape[0]
    C = ah1_ref.shape[1]
    G = GC // C
    Do = w2_ref.shape[1]
    f32 = jnp.float32
    cdt = jnp.bfloat16

    @pl.when(pl.program_id(1) == 0)
    def _prep():
        w1s[...] = w1_ref[...].astype(cdt)
        w2s[...] = w2_ref[...].astype(cdt)
        fcws[...] = fcw_ref[...].astype(cdt)

    x = x_ref[...].astype(cdt)

    s1 = jnp.dot(x, w1s[...], preferred_element_type=f32)
    s1 = s1.astype(cdt)

    a1 = ah1_ref[...]
    h1 = jnp.concatenate(
        [jnp.dot(a1, s1[g * C:(g + 1) * C], preferred_element_type=f32)
         for g in range(G)], axis=0)
    h1 = jnp.maximum(h1, 0.2 * h1)

    s2 = jnp.dot(h1.astype(cdt), w2s[...],
                 preferred_element_type=f32).astype(cdt)

    a2 = ah2_ref[...]
    f1s, f2s = [], []
    for g in range(G):
        gg = jnp.dot(a2, s2[g * 2 * C:(g + 1) * 2 * C],
                     preferred_element_type=f32)
        f1s.append(gg[:C])
        f2s.append(gg[C:])
    f1 = jnp.concatenate(f1s, axis=0).astype(cdt)
    f2 = jnp.concatenate(f2s, axis=0).astype(cdt)

    dn = (((1,), (1,)), ((), ()))
    fcw = fcws[...]
    pre = (jax.lax.dot_general(f1, fcw[:, 0:Do], dn, preferred_element_type=f32)
           + jax.lax.dot_general(f2, fcw[:, Do:2 * Do], dn, preferred_element_type=f32)
           + jax.lax.dot_general(x, fcw[:, 2 * Do:], dn, preferred_element_type=f32)
           + fcb_ref[...])
    out = jnp.tanh(pre)

    out_ref[...] = (jnp.sum(out * clsw_ref[...], axis=-1, keepdims=True)
                    + clsb_ref[...])


def kernel(x_input, semantic_feature, adj1, adj2, gc_w1, gc_w2,
           fc_w, fc_b, cls_w, cls_b):
    B, C, D = semantic_feature.shape
    Dm = gc_w1.shape[1]
    Do = gc_w2.shape[1]
    Dout = fc_w.shape[0]
    G = _G
    S = _SPLIT
    J = B // G // S
    f32 = jnp.float32
    cdt = jnp.bfloat16

    sem_flat = semantic_feature.reshape(B * C, D)

    a1f = adj1.astype(f32)
    a2f = adj2.astype(f32)
    a_hop1 = jnp.concatenate([a1f, a2f], axis=0).astype(cdt)
    zeros = jnp.zeros((C, C), f32)
    a_hop2 = jnp.concatenate(
        [jnp.concatenate([a1f, zeros], axis=1),
         jnp.concatenate([zeros, a2f], axis=1)], axis=0).astype(cdt)

    fcb = fc_b.reshape(1, Dout)
    clsw = jnp.tile(cls_w, (G, 1))
    clsb = jnp.tile(cls_b.reshape(C, 1), (G, 1))

    out = pl.pallas_call(
        _gcn_kernel,
        out_shape=jax.ShapeDtypeStruct((B * C, 1), f32),
        grid_spec=pltpu.PrefetchScalarGridSpec(
            num_scalar_prefetch=0,
            grid=(S, J),
            in_specs=[
                pl.BlockSpec((G * C, D), lambda i, j: (i * J + j, 0)),
                pl.BlockSpec((2 * C, C), lambda i, j: (0, 0)),
                pl.BlockSpec((2 * C, 2 * C), lambda i, j: (0, 0)),
                pl.BlockSpec((D, Dm), lambda i, j: (0, 0)),
                pl.BlockSpec((Dm, Do), lambda i, j: (0, 0)),
                pl.BlockSpec((Dout, 2 * Do + D), lambda i, j: (0, 0)),
                pl.BlockSpec((1, Dout), lambda i, j: (0, 0)),
                pl.BlockSpec((G * C, Dout), lambda i, j: (0, 0)),
                pl.BlockSpec((G * C, 1), lambda i, j: (0, 0)),
            ],
            out_specs=pl.BlockSpec((G * C, 1), lambda i, j: (i * J + j, 0)),
            scratch_shapes=[
                pltpu.VMEM((D, Dm), cdt),
                pltpu.VMEM((Dm, Do), cdt),
                pltpu.VMEM((Dout, 2 * Do + D), cdt),
            ],
        ),
        compiler_params=pltpu.CompilerParams(
            dimension_semantics=("parallel", "arbitrary"),
            vmem_limit_bytes=60 << 20,
        ),
    )(sem_flat, a_hop1, a_hop2, gc_w1, gc_w2, fc_w, fcb, clsw, clsb)

    return out.reshape(B, C)
